# baseline (speedup 1.0000x reference)
import functools

import jax
import jax.numpy as jnp
from jax import lax
from jax.experimental import pallas as pl
from jax.experimental.pallas import tpu as pltpu
from jax.experimental.pallas import tpu_sc as plsc

N = 10000
E = 320000
D = 128
NP = 10240
NCORES = 2
NSUB = 16
NW = NCORES * NSUB
EPT = 10240
EPAD = EPT * NW
C = 64
NCHUNK = EPT // C
ROWS_PER_TILE = NP // NSUB


def _lorentz_sq(x):
    full = jnp.sum(x * x, axis=-1, keepdims=True)
    return full - 2.0 * x[:, :1] * x[:, :1]


def _norm_factor(x):
    return jnp.sqrt(jnp.clip(jnp.abs(_lorentz_sq(x)), 1e-8, None))


def _pre_body(xs_ref, xh_ref, wq_ref, wk_ref, wv_ref, ws_ref,
              a_ref, b_ref, v_ref, bm_ref):
    xs = xs_ref[...]
    xh = xh_ref[...]
    wq = wq_ref[...]
    wk = wk_ref[...]
    wv = wv_ref[...]
    ws = ws_ref[...]

    qp = jnp.dot(xs, wq.T, preferred_element_type=jnp.float32)
    kp = jnp.dot(xh, wk.T, preferred_element_type=jnp.float32)
    vp = jnp.dot(xh, wv.T, preferred_element_type=jnp.float32)

    q = qp / _norm_factor(qp)
    k = kp / _norm_factor(kp)
    v = vp / _norm_factor(vp)

    w1 = ws[0:1, 0:D]
    w2 = ws[0:1, D:2 * D]
    a_ref[...] = jnp.sum(q * w1, axis=-1, keepdims=True)
    bcol = jnp.sum(k * w2, axis=-1, keepdims=True)
    b_ref[...] = bcol
    v_ref[...] = v

    bb = jnp.max(bcol).reshape(1, 1)

    @pl.when(pl.program_id(0) == 0)
    def _():
        bm_ref[...] = bb

    @pl.when(pl.program_id(0) > 0)
    def _():
        bm_ref[...] = jnp.maximum(bm_ref[...], bb)


def _pre(x_S, x_H, W_q, W_k, W_v, W_s):
    blk = 1000
    grid = N // blk
    return pl.pallas_call(
        _pre_body,
        grid=(grid,),
        in_specs=[
            pl.BlockSpec((blk, D), lambda i: (i, 0)),
            pl.BlockSpec((blk, D), lambda i: (i, 0)),
            pl.BlockSpec((D, D), lambda i: (0, 0)),
            pl.BlockSpec((D, D), lambda i: (0, 0)),
            pl.BlockSpec((D, D), lambda i: (0, 0)),
            pl.BlockSpec((1, 2 * D), lambda i: (0, 0)),
        ],
        out_specs=[
            pl.BlockSpec((blk, 1), lambda i: (i, 0)),
            pl.BlockSpec((blk, 1), lambda i: (i, 0)),
            pl.BlockSpec((blk, D), lambda i: (i, 0)),
            pl.BlockSpec((1, 1), lambda i: (0, 0)),
        ],
        out_shape=[
            jax.ShapeDtypeStruct((N, 1), jnp.float32),
            jax.ShapeDtypeStruct((N, 1), jnp.float32),
            jax.ShapeDtypeStruct((N, D), jnp.float32),
            jax.ShapeDtypeStruct((1, 1), jnp.float32),
        ],
    )(x_S, x_H, W_q, W_k, W_v, W_s)


def _sc_body(a_hbm, b_hbm, bm_hbm, v_hbm, ed_hbm,
             numer_hbm, denom_hbm,
             at, bt, bmv, idxps, exbs, rowss,
             nsh, dsh, semis, semgs):
    cid = lax.axis_index("c")
    sid = lax.axis_index("s")
    wid = cid * NSUB + sid

    zero16 = jnp.zeros((16,), jnp.float32)

    pltpu.sync_copy(a_hbm, at)
    pltpu.sync_copy(b_hbm, bt)
    pltpu.sync_copy(bm_hbm, bmv)

    rows0 = rowss[0]

    def _zrow(r, _):
        for j in range(8):
            rows0[r, pl.ds(j * 16, 16)] = zero16
        return 0
    lax.fori_loop(0, C, _zrow, 0)

    for g in range(C // 16):
        exbs[0][pl.ds(g * 16, 16)] = zero16

    nbase = sid * ROWS_PER_TILE
    for t in range(ROWS_PER_TILE // C):
        pltpu.sync_copy(rows0, nsh.at[pl.ds(nbase + t * C, C)])
        pltpu.sync_copy(exbs[0], dsh.at[pl.ds(nbase + t * C, C)])
    plsc.subcore_barrier()

    bmax = bmv[pl.ds(0, 16)]

    cbase = wid * NCHUNK

    def _idx_start(i, bi):
        pltpu.async_copy(ed_hbm.at[cbase + i], idxps[bi], semis[bi])

    def _idx_wait(bi):
        pltpu.make_async_copy(ed_hbm.at[cbase], idxps[bi], semis[bi]).wait()

    def _gather(bi, r):
        pltpu.async_copy(v_hbm.at[idxps[bi].at[1]], rowss[r], semgs[r])

    def _gather_wait(bi, r):
        pltpu.make_async_copy(
            v_hbm.at[idxps[bi].at[1]], rowss[r], semgs[r]).wait()

    def _scatter(bi, r):
        pltpu.async_copy(rowss[r], nsh.at[idxps[bi].at[0]], semgs[r],
                         add=True)
        pltpu.async_copy(exbs[r], dsh.at[idxps[bi].at[0]], semgs[r],
                         add=True)

    def _scatter_wait(bi, r):
        pltpu.make_async_copy(
            rowss[r], nsh.at[idxps[bi].at[0]], semgs[r]).wait()
        pltpu.make_async_copy(
            exbs[r], dsh.at[idxps[bi].at[0]], semgs[r]).wait()

    def _compute(bi, r):
        idx_b = idxps[bi]
        exb_b = exbs[r]
        rows_b = rowss[r]
        for g in range(C // 16):
            s16 = idx_b[0, pl.ds(g * 16, 16)]
            d16 = idx_b[1, pl.ds(g * 16, 16)]
            av = plsc.load_gather(at, [s16])
            bv = plsc.load_gather(bt, [d16])
            x = av + bv
            sc = jnp.maximum(x, 0.01 * x)
            xm = av + bmax
            cm = jnp.maximum(xm, 0.01 * xm)
            ex = jnp.exp(sc - cm)
            exb_b[pl.ds(g * 16, 16)] = ex

        def _scale(r4, _):
            for u in range(4):
                rr = r4 * 4 + u
                w = plsc.load_gather(exb_b, [jnp.broadcast_to(rr, (16,))])
                for j in range(8):
                    rows_b[rr, pl.ds(j * 16, 16)] = (
                        rows_b[rr, pl.ds(j * 16, 16)] * w)
            return 0
        lax.fori_loop(0, C // 4, _scale, 0)

    for k in range(4):
        _idx_start(k, k)
    _idx_wait(0)
    _gather(0, 0)
    _idx_wait(1)
    _gather(1, 1)

    def _body(i, r, bi, first, dynamic):
        _gather_wait(bi, r)
        _compute(bi, r)
        _scatter(bi, r)
        r2 = (r + 2) % 3
        bi2 = (bi + 2) % 6

        def _pf():
            if not first:
                _scatter_wait((bi2 + 6 - 3) % 6, r2)
            _idx_wait(bi2)
            _gather(bi2, r2)

        def _is():
            _idx_start(i + 4, (bi + 4) % 6)

        if dynamic:
            pl.when(i + 2 < NCHUNK)(_pf)
            pl.when(i + 4 < NCHUNK)(_is)
        else:
            if i + 2 < NCHUNK:
                _pf()
            if i + 4 < NCHUNK:
                _is()

    _body(0, 0, 0, True, False)
    _body(1, 1, 1, False, False)

    def _outer(io, _):
        for b in range(6):
            i = 2 + io * 6 + b
            _body(i, (2 + b) % 3, (2 + b) % 6, False, True)
        return 0

    steady = (NCHUNK - 2) // 6
    lax.fori_loop(0, steady, _outer, 0)
    for k in range(NCHUNK - 2 - steady * 6):
        i = 2 + steady * 6 + k
        _body(i, i % 3, i % 6, False, False)

    for i in range(NCHUNK - 3, NCHUNK):
        _scatter_wait(i % 6, i % 3)
    plsc.subcore_barrier()

    pltpu.sync_copy(dsh.at[pl.ds(nbase, ROWS_PER_TILE)],
                    denom_hbm.at[cid, pl.ds(nbase, ROWS_PER_TILE)])
    pltpu.sync_copy(nsh.at[pl.ds(nbase, ROWS_PER_TILE)],
                    numer_hbm.at[cid, pl.ds(nbase, ROWS_PER_TILE)])


@functools.partial(
    pl.kernel,
    out_type=[
        jax.ShapeDtypeStruct((NCORES, NP, D), jnp.float32),
        jax.ShapeDtypeStruct((NCORES, NP), jnp.float32),
    ],
    mesh=plsc.VectorSubcoreMesh(core_axis_name="c", subcore_axis_name="s"),
    compiler_params=pltpu.CompilerParams(needs_layout_passes=False),
    scratch_types=[
        pltpu.VMEM((NP,), jnp.float32),
        pltpu.VMEM((NP,), jnp.float32),
        pltpu.VMEM((16,), jnp.float32),
        [pltpu.VMEM((2, C), jnp.int32) for _ in range(6)],
        [pltpu.VMEM((C,), jnp.float32) for _ in range(3)],
        [pltpu.VMEM((C, D), jnp.float32) for _ in range(3)],
        pltpu.VMEM_SHARED((NP, D), jnp.float32),
        pltpu.VMEM_SHARED((NP,), jnp.float32),
        [pltpu.SemaphoreType.DMA for _ in range(6)],
        [pltpu.SemaphoreType.DMA for _ in range(3)],
    ],
)
def _edge_sc(a_hbm, b_hbm, bm_hbm, v_hbm, ed_hbm,
             numer_hbm, denom_hbm,
             at, bt, bmv, idxps, exbs, rowss,
             nsh, dsh, semis, semgs):
    _sc_body(a_hbm, b_hbm, bm_hbm, v_hbm, ed_hbm,
             numer_hbm, denom_hbm,
             at, bt, bmv, idxps, exbs, rowss,
             nsh, dsh, semis, semgs)


def _post_body(n0_ref, n1_ref, d0_ref, d1_ref, xh_ref, wp_ref, z_ref):
    agg = n0_ref[...] + n1_ref[...]
    d = d0_ref[...] + d1_ref[...]
    inv = jnp.where(d > 0.0, 1.0 / d, 0.0)
    out = agg * inv
    col = lax.broadcasted_iota(jnp.int32, out.shape, 1)
    out = out + jnp.where(col == 0, 1.0, 0.0)
    out = out / _norm_factor(out)
    op = jnp.dot(out, wp_ref[...].T, preferred_element_type=jnp.float32)
    op = op / _norm_factor(op)
    s = op + xh_ref[...]
    z_ref[...] = s / _norm_factor(s)


def _post(n0, n1, d0, d1, x_H, W_p):
    blk = 1000
    grid = N // blk
    return pl.pallas_call(
        _post_body,
        grid=(grid,),
        in_specs=[
            pl.BlockSpec((blk, D), lambda i: (i, 0)),
            pl.BlockSpec((blk, D), lambda i: (i, 0)),
            pl.BlockSpec((blk, 1), lambda i: (i, 0)),
            pl.BlockSpec((blk, 1), lambda i: (i, 0)),
            pl.BlockSpec((blk, D), lambda i: (i, 0)),
            pl.BlockSpec((D, D), lambda i: (0, 0)),
        ],
        out_specs=pl.BlockSpec((blk, D), lambda i: (i, 0)),
        out_shape=jax.ShapeDtypeStruct((N, D), jnp.float32),
    )(n0, n1, d0, d1, x_H, W_p)


@jax.jit
def kernel(x_H, x_S, edge_index, W_q, W_k, W_v, W_s, W_p):
    src = edge_index[0].astype(jnp.int32)
    dst = edge_index[1].astype(jnp.int32)
    pad = jnp.full((EPAD - E,), N, jnp.int32)
    srcp = jnp.concatenate([src, pad])
    dstp = jnp.concatenate([dst, pad])
    ed = jnp.stack([srcp.reshape(EPAD // C, C), dstp.reshape(EPAD // C, C)],
                   axis=1)
    a2, b2, v, bm = _pre(x_S, x_H, W_q, W_k, W_v, W_s)
    zpad = jnp.zeros((NP - N,), jnp.float32)
    ap = jnp.concatenate([a2[:, 0], zpad])
    bp = jnp.concatenate([b2[:, 0], zpad])
    vp = jnp.concatenate([v, jnp.zeros((NP - N, D), jnp.float32)])
    bvec = jnp.broadcast_to(bm[0], (16,))
    numer, denom = _edge_sc(ap, bp, bvec, vp, ed)
    z = _post(numer[0, :N], numer[1, :N],
              denom[0, :N, None], denom[1, :N, None], x_H, W_p)
    return z


# speedup vs baseline: 1.7707x; 1.7707x over previous
import functools

import jax
import jax.numpy as jnp
from jax import lax
from jax.experimental import pallas as pl
from jax.experimental.pallas import tpu as pltpu
from jax.experimental.pallas import tpu_sc as plsc

N = 10000
E = 320000
D = 128
NP = 10240
NCORES = 2
NSUB = 16
NW = NCORES * NSUB
EPT = E // NW
C = 80
NCHUNK = EPT // C
ROWS_PER_TILE = NP // NSUB


def _lorentz_sq(x):
    full = jnp.sum(x * x, axis=-1, keepdims=True)
    return full - 2.0 * x[:, :1] * x[:, :1]


def _norm_factor(x):
    return jnp.sqrt(jnp.clip(jnp.abs(_lorentz_sq(x)), 1e-8, None))


def _pre_body(xs_ref, xh_ref, wq_ref, wk_ref, wv_ref, ws_ref,
              a_ref, b_ref, v_ref, bm_ref):
    xs = xs_ref[...]
    xh = xh_ref[...]
    wq = wq_ref[...]
    wk = wk_ref[...]
    wv = wv_ref[...]
    ws = ws_ref[...]

    qp = jnp.dot(xs, wq.T, preferred_element_type=jnp.float32)
    kp = jnp.dot(xh, wk.T, preferred_element_type=jnp.float32)
    vp = jnp.dot(xh, wv.T, preferred_element_type=jnp.float32)

    q = qp / _norm_factor(qp)
    k = kp / _norm_factor(kp)
    v = vp / _norm_factor(vp)

    w1 = ws[0:1, 0:D]
    w2 = ws[0:1, D:2 * D]
    a_ref[...] = jnp.sum(q * w1, axis=-1, keepdims=True)
    bcol = jnp.sum(k * w2, axis=-1, keepdims=True)
    b_ref[...] = bcol
    v_ref[...] = v

    bb = jnp.max(bcol).reshape(1, 1)

    @pl.when(pl.program_id(0) == 0)
    def _():
        bm_ref[...] = bb

    @pl.when(pl.program_id(0) > 0)
    def _():
        bm_ref[...] = jnp.maximum(bm_ref[...], bb)


def _pre(x_S, x_H, W_q, W_k, W_v, W_s):
    blk = 1000
    grid = N // blk
    return pl.pallas_call(
        _pre_body,
        grid=(grid,),
        in_specs=[
            pl.BlockSpec((blk, D), lambda i: (i, 0)),
            pl.BlockSpec((blk, D), lambda i: (i, 0)),
            pl.BlockSpec((D, D), lambda i: (0, 0)),
            pl.BlockSpec((D, D), lambda i: (0, 0)),
            pl.BlockSpec((D, D), lambda i: (0, 0)),
            pl.BlockSpec((1, 2 * D), lambda i: (0, 0)),
        ],
        out_specs=[
            pl.BlockSpec((blk, 1), lambda i: (i, 0)),
            pl.BlockSpec((blk, 1), lambda i: (i, 0)),
            pl.BlockSpec((blk, D), lambda i: (i, 0)),
            pl.BlockSpec((1, 1), lambda i: (0, 0)),
        ],
        out_shape=[
            jax.ShapeDtypeStruct((N, 1), jnp.float32),
            jax.ShapeDtypeStruct((N, 1), jnp.float32),
            jax.ShapeDtypeStruct((N, D), jnp.float32),
            jax.ShapeDtypeStruct((1, 1), jnp.float32),
        ],
    )(x_S, x_H, W_q, W_k, W_v, W_s)


def _sc_body(a_hbm, b_hbm, bm_hbm, v_hbm, ed_hbm,
             numer_hbm, denom_hbm,
             at, bt, bmv, idxp0, idxp1, idxp2, idxp3,
             exb0, rows0, exb1, rows1,
             acc, nsh, dsh, semi0, semi1, semi2, semi3, semg0, semg1):
    cid = lax.axis_index("c")
    sid = lax.axis_index("s")
    wid = cid * NSUB + sid

    zero16 = jnp.zeros((16,), jnp.float32)

    pltpu.sync_copy(a_hbm, at)
    pltpu.sync_copy(b_hbm, bt)
    pltpu.sync_copy(bm_hbm, bmv)

    def _zrow(r, _):
        for j in range(8):
            rows0[r, pl.ds(j * 16, 16)] = zero16
        return 0
    lax.fori_loop(0, C, _zrow, 0)

    def _zacc(i, _):
        acc[pl.ds(i * 16, 16)] = zero16
        return 0
    lax.fori_loop(0, ROWS_PER_TILE // 16, _zacc, 0)

    nbase = sid * ROWS_PER_TILE
    for t in range(ROWS_PER_TILE // C):
        pltpu.sync_copy(rows0, nsh.at[pl.ds(nbase + t * C, C)])
    pltpu.sync_copy(acc, dsh.at[pl.ds(nbase, ROWS_PER_TILE)])
    plsc.subcore_barrier()

    bmax = bmv[pl.ds(0, 16)]

    cbase = wid * NCHUNK
    ibufs = [(idxp0, semi0), (idxp1, semi1), (idxp2, semi2), (idxp3, semi3)]
    rbufs = [(exb0, rows0, semg0), (exb1, rows1, semg1)]

    def _idx_start(i, bi):
        idx_b, semi_b = ibufs[bi]
        pltpu.async_copy(ed_hbm.at[cbase + i], idx_b, semi_b)

    def _idx_wait(bi):
        idx_b, semi_b = ibufs[bi]
        pltpu.make_async_copy(ed_hbm.at[cbase], idx_b, semi_b).wait()

    def _gather(bi, br):
        idx_b, _ = ibufs[bi]
        _, rows_b, semg_b = rbufs[br]
        pltpu.async_copy(v_hbm.at[idx_b.at[1]], rows_b, semg_b)

    def _step(i, bi, br):
        idx_b, _ = ibufs[bi]
        exb_b, rows_b, semg_b = rbufs[br]
        pltpu.make_async_copy(v_hbm.at[idx_b.at[1]], rows_b, semg_b).wait()
        for g in range(C // 16):
            s16 = idx_b[0, pl.ds(g * 16, 16)]
            d16 = idx_b[1, pl.ds(g * 16, 16)]
            av = plsc.load_gather(at, [s16])
            bv = plsc.load_gather(bt, [d16])
            x = av + bv
            sc = jnp.maximum(x, 0.01 * x)
            xm = av + bmax
            cm = jnp.maximum(xm, 0.01 * xm)
            ex = jnp.exp(sc - cm)
            exb_b[pl.ds(g * 16, 16)] = ex

        def _scale(r4, _):
            for u in range(4):
                r = r4 * 4 + u
                w = plsc.load_gather(exb_b, [jnp.broadcast_to(r, (16,))])
                for j in range(8):
                    rows_b[r, pl.ds(j * 16, 16)] = (
                        rows_b[r, pl.ds(j * 16, 16)] * w)
            return 0
        lax.fori_loop(0, C // 4, _scale, 0)

        pltpu.sync_copy(rows_b, nsh.at[idx_b.at[0]], add=True)
        pltpu.sync_copy(exb_b, dsh.at[idx_b.at[0]], add=True)

        @pl.when(i + 2 < NCHUNK)
        def _():
            _idx_wait((bi + 2) % 4)
            _gather((bi + 2) % 4, br)

        @pl.when(i + 4 < NCHUNK)
        def _():
            _idx_start(i + 4, bi)

    for k in range(4):
        _idx_start(k, k)
    _idx_wait(0)
    _gather(0, 0)
    _idx_wait(1)
    _gather(1, 1)

    def _outer(io, _):
        for b in range(4):
            _step(io * 4 + b, b, b % 2)
        return 0

    lax.fori_loop(0, (NCHUNK - 1) // 4, _outer, 0)
    _step(NCHUNK - 1, (NCHUNK - 1) % 4, (NCHUNK - 1) % 2)
    plsc.subcore_barrier()

    pltpu.sync_copy(dsh.at[pl.ds(nbase, ROWS_PER_TILE)],
                    denom_hbm.at[cid, pl.ds(nbase, ROWS_PER_TILE)])
    pltpu.sync_copy(nsh.at[pl.ds(nbase, ROWS_PER_TILE)],
                    numer_hbm.at[cid, pl.ds(nbase, ROWS_PER_TILE)])


@functools.partial(
    pl.kernel,
    out_type=[
        jax.ShapeDtypeStruct((NCORES, NP, D), jnp.float32),
        jax.ShapeDtypeStruct((NCORES, NP), jnp.float32),
    ],
    mesh=plsc.VectorSubcoreMesh(core_axis_name="c", subcore_axis_name="s"),
    compiler_params=pltpu.CompilerParams(needs_layout_passes=False),
    scratch_types=[
        pltpu.VMEM((N,), jnp.float32),
        pltpu.VMEM((N,), jnp.float32),
        pltpu.VMEM((16,), jnp.float32),
        pltpu.VMEM((2, C), jnp.int32),
        pltpu.VMEM((2, C), jnp.int32),
        pltpu.VMEM((2, C), jnp.int32),
        pltpu.VMEM((2, C), jnp.int32),
        pltpu.VMEM((C,), jnp.float32),
        pltpu.VMEM((C, D), jnp.float32),
        pltpu.VMEM((C,), jnp.float32),
        pltpu.VMEM((C, D), jnp.float32),
        pltpu.VMEM((ROWS_PER_TILE,), jnp.float32),
        pltpu.VMEM_SHARED((NP, D), jnp.float32),
        pltpu.VMEM_SHARED((NP,), jnp.float32),
        pltpu.SemaphoreType.DMA,
        pltpu.SemaphoreType.DMA,
        pltpu.SemaphoreType.DMA,
        pltpu.SemaphoreType.DMA,
        pltpu.SemaphoreType.DMA,
        pltpu.SemaphoreType.DMA,
    ],
)
def _edge_sc(a_hbm, b_hbm, bm_hbm, v_hbm, ed_hbm,
             numer_hbm, denom_hbm,
             at, bt, bmv, idxp0, idxp1, idxp2, idxp3,
             exb0, rows0, exb1, rows1,
             acc, nsh, dsh, semi0, semi1, semi2, semi3, semg0, semg1):
    _sc_body(a_hbm, b_hbm, bm_hbm, v_hbm, ed_hbm,
             numer_hbm, denom_hbm,
             at, bt, bmv, idxp0, idxp1, idxp2, idxp3,
             exb0, rows0, exb1, rows1,
             acc, nsh, dsh, semi0, semi1, semi2, semi3, semg0, semg1)


def _post_body(n0_ref, n1_ref, d0_ref, d1_ref, xh_ref, wp_ref, z_ref):
    agg = n0_ref[...] + n1_ref[...]
    d = d0_ref[...] + d1_ref[...]
    inv = jnp.where(d > 0.0, 1.0 / d, 0.0)
    out = agg * inv
    col = lax.broadcasted_iota(jnp.int32, out.shape, 1)
    out = out + jnp.where(col == 0, 1.0, 0.0)
    out = out / _norm_factor(out)
    op = jnp.dot(out, wp_ref[...].T, preferred_element_type=jnp.float32)
    op = op / _norm_factor(op)
    s = op + xh_ref[...]
    z_ref[...] = s / _norm_factor(s)


def _post(n0, n1, d0, d1, x_H, W_p):
    blk = 1000
    grid = N // blk
    return pl.pallas_call(
        _post_body,
        grid=(grid,),
        in_specs=[
            pl.BlockSpec((blk, D), lambda i: (i, 0)),
            pl.BlockSpec((blk, D), lambda i: (i, 0)),
            pl.BlockSpec((blk, 1), lambda i: (i, 0)),
            pl.BlockSpec((blk, 1), lambda i: (i, 0)),
            pl.BlockSpec((blk, D), lambda i: (i, 0)),
            pl.BlockSpec((D, D), lambda i: (0, 0)),
        ],
        out_specs=pl.BlockSpec((blk, D), lambda i: (i, 0)),
        out_shape=jax.ShapeDtypeStruct((N, D), jnp.float32),
    )(n0, n1, d0, d1, x_H, W_p)


@jax.jit
def kernel(x_H, x_S, edge_index, W_q, W_k, W_v, W_s, W_p):
    src = edge_index[0].astype(jnp.int32)
    dst = edge_index[1].astype(jnp.int32)
    ed = jnp.stack([src.reshape(E // C, C), dst.reshape(E // C, C)], axis=1)
    a2, b2, v, bm = _pre(x_S, x_H, W_q, W_k, W_v, W_s)
    bvec = jnp.broadcast_to(bm[0], (16,))
    numer, denom = _edge_sc(a2[:, 0], b2[:, 0], bvec, v, ed)
    z = _post(numer[0, :N], numer[1, :N],
              denom[0, :N, None], denom[1, :N, None], x_H, W_p)
    return z
